# Initial kernel scaffold; baseline (speedup 1.0000x reference)
import jax
import jax.numpy as jnp
from jax.experimental import pallas as pl


def kernel(x, src, tgt, Wf, bf, Ww, bw):
    raise NotImplementedError("write your pallas kernel here")


# speedup vs baseline: 1.6885x; 1.6885x over previous
import dataclasses

import jax
import jax.numpy as jnp
from jax import lax
from jax.experimental import pallas as pl
from jax.experimental.pallas import tpu as pltpu
from jax.experimental.pallas import tpu_sc as plsc

N = 10000
E = 320000
D = 128
EPS = 1e-6

NC = 2
NS = 16
L = 16
EPW = E // NS
B = 80
NCHUNK = EPW // B
DROWS = 80
HALF = N // NC
ACC_ROWS = HALF + 8
DCH = 16
ER = E // D
SHIFT = 14
MASK = (1 << SHIFT) - 1


def _prep_body(x_ref, wf_ref, bf_ref, ws_ref, wt_ref, src_ref, tgt_ref,
               xcat_ref, es_ref, et_ref, edges_ref):
    x = x_ref[...]
    wf = wf_ref[...]
    dn = (((1,), (1,)), ((), ()))
    dnc = (((1,), (0,)), ((), ()))
    xcat_ref[pl.ds(0, N), :] = lax.dot_general(
        x, wf[:, :D], dn, preferred_element_type=jnp.float32)
    xcat_ref[pl.ds(N, N), :] = lax.dot_general(
        x, wf[:, D:], dn, preferred_element_type=jnp.float32) + bf_ref[...][None, :]
    a_s8 = lax.dot_general(x, ws_ref[...], dnc, preferred_element_type=jnp.float32)
    a_t8 = lax.dot_general(x, wt_ref[...], dnc, preferred_element_type=jnp.float32)
    es_ref[...] = jnp.exp(a_s8[:, :1] - jnp.max(a_s8))
    et_ref[...] = jnp.exp(a_t8[:, :1] - jnp.max(a_t8))
    edges_ref[...] = jnp.bitwise_or(
        jnp.left_shift(src_ref[...], SHIFT), tgt_ref[...])


def _prep(x, Wf, bf, ws_pad, wt_pad, src2d, tgt2d):
    return pl.pallas_call(
        _prep_body,
        out_shape=[
            jax.ShapeDtypeStruct((2 * N, D), jnp.float32),
            jax.ShapeDtypeStruct((N, 1), jnp.float32),
            jax.ShapeDtypeStruct((N, 1), jnp.float32),
            jax.ShapeDtypeStruct((ER, D), jnp.int32),
        ],
    )(x, Wf, bf, ws_pad, wt_pad, src2d, tgt2d)


def _edge_body(xcat_hbm, es_hbm, edges_hbm, acc_hbm, den_hbm,
               ebuf_v, idx_v, tgt_v, tcl_v, st_v, o_v, es_v, den_v,
               acc_sh, sem):
    cid = lax.axis_index("c")
    sid = lax.axis_index("s")

    lanes = lax.iota(jnp.int32, L)
    lane0 = lanes == 0
    nbase = cid * HALF

    @pl.loop(0, DROWS)
    def _zden(r):
        for j in range(D // L):
            den_v[r, pl.ds(j * L, L)] = jnp.zeros((L,), jnp.float32)

    @pl.loop(0, B)
    def _zo(r):
        for j in range(D // L):
            o_v[r, pl.ds(j * L, L)] = jnp.zeros((L,), jnp.float32)

    @pl.loop(sid, ACC_ROWS // DCH, step=NS)
    def _zacc(k):
        pltpu.sync_copy(o_v.at[pl.ds(0, DCH)], acc_sh.at[pl.ds(k * DCH, DCH)])

    pltpu.sync_copy(es_hbm, es_v)
    plsc.subcore_barrier()

    @pl.loop(0, NCHUNK)
    def _chunk(i):
        base = sid * EPW + i * B
        pltpu.sync_copy(edges_hbm.at[pl.ds(base, B)], ebuf_v)

        @pl.loop(0, B // L)
        def _mkidx(g):
            p16 = ebuf_v[pl.ds(g * L, L)]
            t16 = lax.bitwise_and(p16, MASK)
            idx_v[pl.ds(g * L, L)] = lax.shift_right_logical(p16, SHIFT)
            idx_v[pl.ds(B + g * L, L)] = t16 + N
            tgt_v[pl.ds(g * L, L)] = t16
            loc = t16 - nbase
            loc = jnp.where(jnp.logical_and(loc >= 0, loc < HALF), loc, HALF)
            tcl_v[pl.ds(g * L, L)] = loc

        pltpu.async_copy(xcat_hbm.at[idx_v], st_v, sem).wait()

        @pl.loop(0, B)
        def _edge(e):
            ev = jnp.full((L,), e, jnp.int32)
            sb = plsc.load_gather(idx_v, [ev])
            tb = plsc.load_gather(tgt_v, [ev])
            wv = plsc.load_gather(es_v, [sb])
            for j in range(D // L):
                sv = st_v[e, pl.ds(j * L, L)]
                tv = st_v[B + e, pl.ds(j * L, L)]
                o_v[e, pl.ds(j * L, L)] = jnp.maximum(sv + tv, 0.0) * wv
            inr = jnp.logical_and(lane0, jnp.logical_and(
                tb >= nbase, tb < nbase + HALF))
            plsc.addupdate_scatter(den_v, [lax.shift_right_logical(tb, 7),
                                           lax.bitwise_and(tb, 127)],
                                   wv, mask=inr)

        pltpu.sync_copy(o_v, acc_sh.at[tcl_v], add=True)

    plsc.subcore_barrier()

    @pl.loop(sid, ACC_ROWS // DCH, step=NS)
    def _drain(k):
        pltpu.sync_copy(acc_sh.at[pl.ds(k * DCH, DCH)],
                        acc_hbm.at[cid, pl.ds(k * DCH, DCH)])

    pltpu.sync_copy(den_v, den_hbm.at[cid, sid])


def _edge(xcat, es, edges):
    mesh = plsc.VectorSubcoreMesh(core_axis_name="c", subcore_axis_name="s")
    cp = pltpu.CompilerParams()
    if "needs_layout_passes" in pltpu.CompilerParams.__dataclass_fields__:
        cp = dataclasses.replace(cp, needs_layout_passes=False)
    f = pl.kernel(
        _edge_body,
        out_type=[
            jax.ShapeDtypeStruct((NC, ACC_ROWS, D), jnp.float32),
            jax.ShapeDtypeStruct((NC, NS, DROWS, D), jnp.float32),
        ],
        mesh=mesh,
        scratch_types=[
            pltpu.VMEM((B,), jnp.int32),
            pltpu.VMEM((2 * B,), jnp.int32),
            pltpu.VMEM((B,), jnp.int32),
            pltpu.VMEM((B,), jnp.int32),
            pltpu.VMEM((2 * B, D), jnp.float32),
            pltpu.VMEM((B, D), jnp.float32),
            pltpu.VMEM((N,), jnp.float32),
            pltpu.VMEM((DROWS, D), jnp.float32),
            pltpu.VMEM_SHARED((ACC_ROWS, D), jnp.float32),
            pltpu.SemaphoreType.DMA,
        ],
        compiler_params=cp,
    )
    return f(xcat, es, edges)


def _final_body(num_ref, dens_ref, et_ref, o_ref):
    den = lax.dot_general(dens_ref[...], jnp.ones((NC * NS, 1), jnp.float32),
                          (((0,), (0,)), ((), ())),
                          preferred_element_type=jnp.float32)[:N]
    et = et_ref[...]
    scale = et / (den * et + EPS)
    o_ref[pl.ds(0, HALF), :] = num_ref[0, :HALF] * scale[:HALF]
    o_ref[pl.ds(HALF, HALF), :] = num_ref[1, :HALF] * scale[HALF:]


def _final(num, dens, et):
    return pl.pallas_call(
        _final_body,
        out_shape=jax.ShapeDtypeStruct((N, D), jnp.float32),
    )(num, dens, et)


def kernel(x, src, tgt, Wf, bf, Ww, bw):
    zpad = jnp.zeros((D, 7), jnp.float32)
    ws_pad = jnp.concatenate([Ww[:, :D].T, zpad], axis=1)
    wt_pad = jnp.concatenate([Ww[:, D:].T, zpad], axis=1)
    xcat, es, et, edges = _prep(x, Wf, bf, ws_pad, wt_pad,
                                src.reshape(ER, D), tgt.reshape(ER, D))
    num, dens = _edge(xcat, es.reshape(N), edges.reshape(E))
    return _final(num, dens.reshape(NC * NS, DROWS * D), et)


# speedup vs baseline: 4.8137x; 2.8509x over previous
import dataclasses

import jax
import jax.numpy as jnp
from jax import lax
from jax.experimental import pallas as pl
from jax.experimental.pallas import tpu as pltpu
from jax.experimental.pallas import tpu_sc as plsc

N = 10000
E = 320000
D = 128
EPS = 1e-6

NC = 2
NS = 16
L = 16
EPW = E // NS
B = 80
NCHUNK = EPW // B
DROWS = 80
HALF = N // NC
ACC_ROWS = HALF + 8
DCH = 16
ER = E // D
SHIFT = 14
MASK = (1 << SHIFT) - 1


def _prep_body(x_ref, wf_ref, bf_ref, ws_ref, wt_ref, src_ref, tgt_ref,
               xcat_ref, es_ref, et_ref, edges_ref):
    x = x_ref[...]
    wf = wf_ref[...]
    dn = (((1,), (1,)), ((), ()))
    dnc = (((1,), (0,)), ((), ()))
    xcat_ref[pl.ds(0, N), :] = lax.dot_general(
        x, wf[:, :D], dn, preferred_element_type=jnp.float32)
    xcat_ref[pl.ds(N, N), :] = lax.dot_general(
        x, wf[:, D:], dn, preferred_element_type=jnp.float32) + bf_ref[...][None, :]
    a_s8 = lax.dot_general(x, ws_ref[...], dnc, preferred_element_type=jnp.float32)
    a_t8 = lax.dot_general(x, wt_ref[...], dnc, preferred_element_type=jnp.float32)
    es_ref[...] = jnp.exp(a_s8[:, :1] - jnp.max(a_s8))
    et_ref[...] = jnp.exp(a_t8[:, :1] - jnp.max(a_t8))
    edges_ref[...] = jnp.bitwise_or(
        jnp.left_shift(src_ref[...], SHIFT), tgt_ref[...])


def _prep(x, Wf, bf, ws_pad, wt_pad, src2d, tgt2d):
    return pl.pallas_call(
        _prep_body,
        out_shape=[
            jax.ShapeDtypeStruct((2 * N, D), jnp.float32),
            jax.ShapeDtypeStruct((N, 1), jnp.float32),
            jax.ShapeDtypeStruct((N, 1), jnp.float32),
            jax.ShapeDtypeStruct((ER, D), jnp.int32),
        ],
    )(x, Wf, bf, ws_pad, wt_pad, src2d, tgt2d)


def _edge_body(xcat_hbm, es_hbm, edges_hbm, acc_hbm, den_hbm,
               ebuf0, ebuf1, idx0, idx1, tgt0, tgt1, tcl0, tcl1,
               st0, st1, es_v, den_v, acc_sh, sem0, sem1):
    cid = lax.axis_index("c")
    sid = lax.axis_index("s")

    lanes = lax.iota(jnp.int32, L)
    lane0 = lanes == 0
    nbase = cid * HALF
    sems = (sem0, sem1)
    ebufs = (ebuf0, ebuf1)
    idxs = (idx0, idx1)
    tgts = (tgt0, tgt1)
    tcls = (tcl0, tcl1)
    sts = (st0, st1)

    @pl.loop(0, DROWS)
    def _zden(r):
        for j in range(D // L):
            den_v[r, pl.ds(j * L, L)] = jnp.zeros((L,), jnp.float32)

    @pl.loop(sid, ACC_ROWS // DCH, step=NS)
    def _zacc(k):
        pltpu.sync_copy(den_v.at[pl.ds(0, DCH)],
                        acc_sh.at[pl.ds(k * DCH, DCH)])

    pltpu.sync_copy(es_hbm, es_v)
    plsc.subcore_barrier()

    def _stage(i, s):
        ebuf_v, idx_v, tgt_v, tcl_v, st_v = (
            ebufs[s], idxs[s], tgts[s], tcls[s], sts[s])
        base = sid * EPW + i * B
        pltpu.sync_copy(edges_hbm.at[pl.ds(base, B)], ebuf_v)

        @pl.loop(0, B // L)
        def _mkidx(g):
            p16 = ebuf_v[pl.ds(g * L, L)]
            t16 = lax.bitwise_and(p16, MASK)
            idx_v[pl.ds(g * L, L)] = lax.shift_right_logical(p16, SHIFT)
            idx_v[pl.ds(B + g * L, L)] = t16 + N
            tgt_v[pl.ds(g * L, L)] = t16
            loc = t16 - nbase
            loc = jnp.where(jnp.logical_and(loc >= 0, loc < HALF), loc, HALF)
            tcl_v[pl.ds(g * L, L)] = loc

        pltpu.async_copy(xcat_hbm.at[idx_v], st_v, sems[s])

    def _work(s):
        idx_v, tgt_v, tcl_v, st_v = idxs[s], tgts[s], tcls[s], sts[s]
        pltpu.make_async_copy(xcat_hbm.at[idx_v], st_v, sems[s]).wait()

        @pl.loop(0, B)
        def _edge(e):
            ev = jnp.full((L,), e, jnp.int32)
            sb = plsc.load_gather(idx_v, [ev])
            tb = plsc.load_gather(tgt_v, [ev])
            wv = plsc.load_gather(es_v, [sb])
            for j in range(D // L):
                sv = st_v[e, pl.ds(j * L, L)]
                tv = st_v[B + e, pl.ds(j * L, L)]
                st_v[e, pl.ds(j * L, L)] = jnp.maximum(sv + tv, 0.0) * wv
            inr = jnp.logical_and(lane0, jnp.logical_and(
                tb >= nbase, tb < nbase + HALF))
            plsc.addupdate_scatter(den_v, [lax.shift_right_logical(tb, 7),
                                           lax.bitwise_and(tb, 127)],
                                   wv, mask=inr)

        pltpu.sync_copy(st_v.at[pl.ds(0, B)], acc_sh.at[tcl_v], add=True)

    _stage(0, 0)

    @pl.loop(0, NCHUNK, step=2)
    def _pair(i):
        _stage(i + 1, 1)
        _work(0)

        @pl.when(i + 2 < NCHUNK)
        def _pre():
            _stage(i + 2, 0)

        _work(1)

    plsc.subcore_barrier()

    @pl.loop(sid, ACC_ROWS // DCH, step=NS)
    def _drain(k):
        pltpu.sync_copy(acc_sh.at[pl.ds(k * DCH, DCH)],
                        acc_hbm.at[cid, pl.ds(k * DCH, DCH)])

    pltpu.sync_copy(den_v, den_hbm.at[cid, sid])


def _edge(xcat, es, edges):
    mesh = plsc.VectorSubcoreMesh(core_axis_name="c", subcore_axis_name="s")
    cp = pltpu.CompilerParams()
    if "needs_layout_passes" in pltpu.CompilerParams.__dataclass_fields__:
        cp = dataclasses.replace(cp, needs_layout_passes=False)
    f = pl.kernel(
        _edge_body,
        out_type=[
            jax.ShapeDtypeStruct((NC, ACC_ROWS, D), jnp.float32),
            jax.ShapeDtypeStruct((NC, NS, DROWS, D), jnp.float32),
        ],
        mesh=mesh,
        scratch_types=[
            pltpu.VMEM((B,), jnp.int32),
            pltpu.VMEM((B,), jnp.int32),
            pltpu.VMEM((2 * B,), jnp.int32),
            pltpu.VMEM((2 * B,), jnp.int32),
            pltpu.VMEM((B,), jnp.int32),
            pltpu.VMEM((B,), jnp.int32),
            pltpu.VMEM((B,), jnp.int32),
            pltpu.VMEM((B,), jnp.int32),
            pltpu.VMEM((2 * B, D), jnp.float32),
            pltpu.VMEM((2 * B, D), jnp.float32),
            pltpu.VMEM((N,), jnp.float32),
            pltpu.VMEM((DROWS, D), jnp.float32),
            pltpu.VMEM_SHARED((ACC_ROWS, D), jnp.float32),
            pltpu.SemaphoreType.DMA,
            pltpu.SemaphoreType.DMA,
        ],
        compiler_params=cp,
    )
    return f(xcat, es, edges)


def _final_body(num_ref, dens_ref, et_ref, o_ref):
    den = lax.dot_general(dens_ref[...], jnp.ones((NC * NS, 1), jnp.float32),
                          (((0,), (0,)), ((), ())),
                          preferred_element_type=jnp.float32)[:N]
    et = et_ref[...]
    scale = et / (den * et + EPS)
    o_ref[pl.ds(0, HALF), :] = num_ref[0, :HALF] * scale[:HALF]
    o_ref[pl.ds(HALF, HALF), :] = num_ref[1, :HALF] * scale[HALF:]


def _final(num, dens, et):
    return pl.pallas_call(
        _final_body,
        out_shape=jax.ShapeDtypeStruct((N, D), jnp.float32),
    )(num, dens, et)


def kernel(x, src, tgt, Wf, bf, Ww, bw):
    zpad = jnp.zeros((D, 7), jnp.float32)
    ws_pad = jnp.concatenate([Ww[:, :D].T, zpad], axis=1)
    wt_pad = jnp.concatenate([Ww[:, D:].T, zpad], axis=1)
    xcat, es, et, edges = _prep(x, Wf, bf, ws_pad, wt_pad,
                                src.reshape(ER, D), tgt.reshape(ER, D))
    num, dens = _edge(xcat, es.reshape(N), edges.reshape(E))
    return _final(num, dens.reshape(NC * NS, DROWS * D), et)


# speedup vs baseline: 4.8572x; 1.0090x over previous
import dataclasses

import jax
import jax.numpy as jnp
from jax import lax
from jax.experimental import pallas as pl
from jax.experimental.pallas import tpu as pltpu
from jax.experimental.pallas import tpu_sc as plsc

N = 10000
E = 320000
D = 128
EPS = 1e-6

NC = 2
NS = 16
L = 16
EPW = E // NS
B = 80
NCHUNK = EPW // B
DROWS = 80
HALF = N // NC
ACC_ROWS = HALF + 8
DCH = 16
ER = E // D
SHIFT = 14
MASK = (1 << SHIFT) - 1


def _prep_body(x_ref, wf_ref, bf_ref, ws_ref, wt_ref, src_ref, tgt_ref,
               xcat_ref, es_ref, et_ref, edges_ref):
    x = x_ref[...]
    wf = wf_ref[...]
    dn = (((1,), (1,)), ((), ()))
    dnc = (((1,), (0,)), ((), ()))
    xcat_ref[pl.ds(0, N), :] = lax.dot_general(
        x, wf[:, :D], dn, preferred_element_type=jnp.float32)
    xcat_ref[pl.ds(N, N), :] = lax.dot_general(
        x, wf[:, D:], dn, preferred_element_type=jnp.float32) + bf_ref[...][None, :]
    a_s8 = lax.dot_general(x, ws_ref[...], dnc, preferred_element_type=jnp.float32)
    a_t8 = lax.dot_general(x, wt_ref[...], dnc, preferred_element_type=jnp.float32)
    es_ref[...] = jnp.exp(a_s8[:, :1] - jnp.max(a_s8))
    et_ref[...] = jnp.exp(a_t8[:, :1] - jnp.max(a_t8))
    edges_ref[...] = jnp.bitwise_or(
        jnp.left_shift(src_ref[...], SHIFT), tgt_ref[...])


def _prep(x, Wf, bf, ws_pad, wt_pad, src2d, tgt2d):
    return pl.pallas_call(
        _prep_body,
        out_shape=[
            jax.ShapeDtypeStruct((2 * N, D), jnp.float32),
            jax.ShapeDtypeStruct((N, 1), jnp.float32),
            jax.ShapeDtypeStruct((N, 1), jnp.float32),
            jax.ShapeDtypeStruct((ER, D), jnp.int32),
        ],
    )(x, Wf, bf, ws_pad, wt_pad, src2d, tgt2d)


def _edge_body(xcat_hbm, es_hbm, edges_hbm, acc_hbm, den_hbm,
               ebuf0, ebuf1, idx0, idx1, tgt0, tgt1, tcl0, tcl1,
               st0, st1, wbuf0, wbuf1, es_v, den_v, acc_sh, sem0, sem1):
    cid = lax.axis_index("c")
    sid = lax.axis_index("s")

    lanes = lax.iota(jnp.int32, L)
    lane0 = lanes == 0
    nbase = cid * HALF
    sems = (sem0, sem1)
    ebufs = (ebuf0, ebuf1)
    idxs = (idx0, idx1)
    tgts = (tgt0, tgt1)
    tcls = (tcl0, tcl1)
    sts = (st0, st1)
    wbufs = (wbuf0, wbuf1)

    @pl.loop(0, DROWS)
    def _zden(r):
        for j in range(D // L):
            den_v[r, pl.ds(j * L, L)] = jnp.zeros((L,), jnp.float32)

    @pl.loop(sid, ACC_ROWS // DCH, step=NS)
    def _zacc(k):
        pltpu.sync_copy(den_v.at[pl.ds(0, DCH)],
                        acc_sh.at[pl.ds(k * DCH, DCH)])

    pltpu.sync_copy(es_hbm, es_v)
    plsc.subcore_barrier()

    def _stage(i, s):
        ebuf_v, idx_v, tgt_v, tcl_v, st_v, wbuf_v = (
            ebufs[s], idxs[s], tgts[s], tcls[s], sts[s], wbufs[s])
        base = sid * EPW + i * B
        pltpu.sync_copy(edges_hbm.at[pl.ds(base, B)], ebuf_v)

        @pl.loop(0, B // L)
        def _mkidx(g):
            p16 = ebuf_v[pl.ds(g * L, L)]
            t16 = lax.bitwise_and(p16, MASK)
            idx_v[pl.ds(g * L, L)] = lax.shift_right_logical(p16, SHIFT)
            idx_v[pl.ds(B + g * L, L)] = t16 + N
            tgt_v[pl.ds(g * L, L)] = t16
            loc = t16 - nbase
            loc = jnp.where(jnp.logical_and(loc >= 0, loc < HALF), loc, HALF)
            tcl_v[pl.ds(g * L, L)] = loc
            wbuf_v[pl.ds(g * L, L)] = plsc.load_gather(
                es_v, [lax.shift_right_logical(p16, SHIFT)])

        pltpu.async_copy(xcat_hbm.at[idx_v], st_v, sems[s])

    def _work(s):
        idx_v, tgt_v, tcl_v, st_v, wbuf_v = (
            idxs[s], tgts[s], tcls[s], sts[s], wbufs[s])
        pltpu.make_async_copy(xcat_hbm.at[idx_v], st_v, sems[s]).wait()

        @pl.loop(0, B)
        def _edge(e):
            ev = jnp.full((L,), e, jnp.int32)
            tb = plsc.load_gather(tgt_v, [ev])
            wv = plsc.load_gather(wbuf_v, [ev])
            for j in range(D // L):
                sv = st_v[e, pl.ds(j * L, L)]
                tv = st_v[B + e, pl.ds(j * L, L)]
                st_v[e, pl.ds(j * L, L)] = jnp.maximum(sv + tv, 0.0) * wv
            inr = jnp.logical_and(lane0, jnp.logical_and(
                tb >= nbase, tb < nbase + HALF))
            plsc.addupdate_scatter(den_v, [lax.shift_right_logical(tb, 7),
                                           lax.bitwise_and(tb, 127)],
                                   wv, mask=inr)

        pltpu.sync_copy(st_v.at[pl.ds(0, B)], acc_sh.at[tcl_v], add=True)

    _stage(0, 0)

    @pl.loop(0, NCHUNK, step=2)
    def _pair(i):
        _stage(i + 1, 1)
        _work(0)

        @pl.when(i + 2 < NCHUNK)
        def _pre():
            _stage(i + 2, 0)

        _work(1)

    plsc.subcore_barrier()

    @pl.loop(sid, ACC_ROWS // DCH, step=NS)
    def _drain(k):
        pltpu.sync_copy(acc_sh.at[pl.ds(k * DCH, DCH)],
                        acc_hbm.at[cid, pl.ds(k * DCH, DCH)])

    pltpu.sync_copy(den_v, den_hbm.at[cid, sid])


def _edge(xcat, es, edges):
    mesh = plsc.VectorSubcoreMesh(core_axis_name="c", subcore_axis_name="s")
    cp = pltpu.CompilerParams()
    if "needs_layout_passes" in pltpu.CompilerParams.__dataclass_fields__:
        cp = dataclasses.replace(cp, needs_layout_passes=False)
    f = pl.kernel(
        _edge_body,
        out_type=[
            jax.ShapeDtypeStruct((NC, ACC_ROWS, D), jnp.float32),
            jax.ShapeDtypeStruct((NC, NS, DROWS, D), jnp.float32),
        ],
        mesh=mesh,
        scratch_types=[
            pltpu.VMEM((B,), jnp.int32),
            pltpu.VMEM((B,), jnp.int32),
            pltpu.VMEM((2 * B,), jnp.int32),
            pltpu.VMEM((2 * B,), jnp.int32),
            pltpu.VMEM((B,), jnp.int32),
            pltpu.VMEM((B,), jnp.int32),
            pltpu.VMEM((B,), jnp.int32),
            pltpu.VMEM((B,), jnp.int32),
            pltpu.VMEM((2 * B, D), jnp.float32),
            pltpu.VMEM((2 * B, D), jnp.float32),
            pltpu.VMEM((B,), jnp.float32),
            pltpu.VMEM((B,), jnp.float32),
            pltpu.VMEM((N,), jnp.float32),
            pltpu.VMEM((DROWS, D), jnp.float32),
            pltpu.VMEM_SHARED((ACC_ROWS, D), jnp.float32),
            pltpu.SemaphoreType.DMA,
            pltpu.SemaphoreType.DMA,
        ],
        compiler_params=cp,
    )
    return f(xcat, es, edges)


def _final_body(num_ref, dens_ref, et_ref, o_ref):
    den = lax.dot_general(dens_ref[...], jnp.ones((NC * NS, 1), jnp.float32),
                          (((0,), (0,)), ((), ())),
                          preferred_element_type=jnp.float32)[:N]
    et = et_ref[...]
    scale = et / (den * et + EPS)
    o_ref[pl.ds(0, HALF), :] = num_ref[0, :HALF] * scale[:HALF]
    o_ref[pl.ds(HALF, HALF), :] = num_ref[1, :HALF] * scale[HALF:]


def _final(num, dens, et):
    return pl.pallas_call(
        _final_body,
        out_shape=jax.ShapeDtypeStruct((N, D), jnp.float32),
    )(num, dens, et)


def kernel(x, src, tgt, Wf, bf, Ww, bw):
    zpad = jnp.zeros((D, 7), jnp.float32)
    ws_pad = jnp.concatenate([Ww[:, :D].T, zpad], axis=1)
    wt_pad = jnp.concatenate([Ww[:, D:].T, zpad], axis=1)
    xcat, es, et, edges = _prep(x, Wf, bf, ws_pad, wt_pad,
                                src.reshape(ER, D), tgt.reshape(ER, D))
    num, dens = _edge(xcat, es.reshape(N), edges.reshape(E))
    return _final(num, dens.reshape(NC * NS, DROWS * D), et)
